# baseline (speedup 1.0000x reference)
import functools

import jax
import jax.numpy as jnp
from jax import lax
from jax.experimental import pallas as pl
from jax.experimental.pallas import tpu as pltpu
from jax.experimental.pallas import tpu_sc as plsc

N = 100000
F = 128
K = 1024

BLK = 2048
NBLK = 49
NPAD = BLK * NBLK
NT = 16
PER_TILE = NPAD // NT
VPT = PER_TILE // 16
WPT = K // NT
MININT = -2**31


def _scores_body(x_ref, m_ref, w_ref, o_ref):
    b = pl.program_id(0)
    w = w_ref[...]
    inv = 1.0 / jnp.sqrt(jnp.sum(w * w))
    s = jnp.dot(x_ref[...], w, preferred_element_type=jnp.float32) * inv
    s = s + m_ref[...]
    row = b * BLK + lax.broadcasted_iota(jnp.int32, (BLK, 1), 0)
    o_ref[...] = jnp.where(row < N, s, -jnp.inf)


_scores_call = pl.pallas_call(
    _scores_body,
    grid=(NBLK,),
    in_specs=[
        pl.BlockSpec((BLK, F), lambda i: (i, 0)),
        pl.BlockSpec((BLK, 1), lambda i: (i, 0)),
        pl.BlockSpec((F, 1), lambda i: (0, 0)),
    ],
    out_specs=pl.BlockSpec((BLK, 1), lambda i: (i, 0)),
    out_shape=jax.ShapeDtypeStruct((NPAD, 1), jnp.float32),
)


_IOTA = lambda: lax.iota(jnp.int32, 16)


def _extract(vec, i):
    return jnp.sum(jnp.where(_IOTA() == i, vec, 0))


def _sel16(vec, kk):
    rv = lax.rev(vec, (0,))
    cs = plsc.cumsum(rv)
    ffs = plsc.all_reduce_ffs(cs >= kk)
    if ffs.ndim:
        ffs = jnp.max(ffs)
    above = _extract(cs, ffs) - _extract(rv, ffs)
    return 15 - ffs, above


def _topk_body(scores_hbm, node_hbm, rows_hbm, wscore_hbm,
               sc_v, key_v, hist_v, ghl_v, ah_v, ghist_v,
               wk_v, wi_v, ws_v, awk_v, awi_v, aws_v,
               cnt_v, acnt_v, rank1_v, myidx_v, mysc_v, rows_v,
               sh_hist, sh_cnt, sh_wk, sh_wi, sh_ws, sem):
    tid = lax.axis_index("s")
    base = tid * PER_TILE
    iota = _IOTA()
    ones = jnp.ones((16,), jnp.int32)
    zeros = jnp.zeros((16,), jnp.int32)
    zerosf = jnp.zeros((16,), jnp.float32)

    pltpu.sync_copy(scores_hbm.at[pl.ds(base, PER_TILE)], sc_v)

    def _mkkeys(j, _):
        v = sc_v[pl.ds(j * 16, 16)]
        b = lax.bitcast_convert_type(v, jnp.int32)
        ku = jnp.where(b < 0, ~b, b ^ MININT)
        key_v[pl.ds(j * 16, 16)] = ku
        return 0

    lax.fori_loop(0, VPT, _mkkeys, 0)

    prefix = jnp.int32(0)
    kk = jnp.int32(K)
    for rnd in range(4):
        shift = 24 - 8 * rnd

        def _zero(g, _):
            hist_v[pl.ds(g * 16, 16)] = zeros
            return 0

        lax.fori_loop(0, 256, _zero, 0)

        pfx_hi = lax.shift_right_logical(prefix, shift + 8) if rnd else None

        def _hist(j, _):
            ku = key_v[pl.ds(j * 16, 16)]
            dig = lax.shift_right_logical(ku, shift) & 0xFF
            hidx = iota * 256 + dig
            if rnd == 0:
                plsc.addupdate_scatter(hist_v, [hidx], ones)
            else:
                m = lax.shift_right_logical(ku, shift + 8) == pfx_hi
                plsc.addupdate_scatter(hist_v, [hidx], ones, mask=m)
            return 0

        lax.fori_loop(0, VPT, _hist, 0)

        def _red(g, _):
            acc = zeros
            for l in range(16):
                acc = acc + hist_v[pl.ds(l * 256 + g * 16, 16)]
            ghl_v[pl.ds(g * 16, 16)] = acc
            return 0

        lax.fori_loop(0, 16, _red, 0)
        pltpu.sync_copy(ghl_v, sh_hist.at[rnd].at[pl.ds(tid * 256, 256)])
        plsc.subcore_barrier()
        pltpu.sync_copy(sh_hist.at[rnd], ah_v)

        def _gred(g, _):
            acc = zeros
            for t in range(16):
                acc = acc + ah_v[pl.ds(t * 256 + g * 16, 16)]
            ghist_v[pl.ds(g * 16, 16)] = acc
            return 0

        lax.fori_loop(0, 16, _gred, 0)

        sgv = zeros
        for g in range(16):
            sgv = sgv + jnp.where(iota == g,
                                  jnp.sum(ghist_v[pl.ds(g * 16, 16)]), 0)
        grp, above_g = _sel16(sgv, kk)
        gvec = ghist_v[pl.ds(grp * 16, 16)]
        dig, above_d = _sel16(gvec, kk - above_g)
        digit = grp * 16 + dig
        prefix = prefix | lax.shift_left(digit, shift)
        kk = kk - above_g - above_d

    thr = prefix
    thr_s = thr ^ MININT

    def _cnt(j, c):
        ku = key_v[pl.ds(j * 16, 16)]
        ks = ku ^ MININT
        cg, ce = c
        cg = cg + jnp.where(ks > thr_s, 1, 0)
        ce = ce + jnp.where(ku == thr, 1, 0)
        return cg, ce

    cgv, cev = lax.fori_loop(0, VPT, _cnt, (zeros, zeros))
    ngt = jnp.sum(cgv)
    neq = jnp.sum(cev)
    cnt_v[...] = jnp.where(iota == 0, ngt, 0) + jnp.where(iota == 1, neq, 0)

    def _zw(z, _):
        wk_v[pl.ds(z * 16, 16)] = zeros
        wi_v[pl.ds(z * 16, 16)] = zeros
        ws_v[pl.ds(z * 16, 16)] = zerosf
        return 0

    lax.fori_loop(0, K // 16, _zw, 0)

    pltpu.sync_copy(cnt_v, sh_cnt.at[pl.ds(tid * 16, 16)])

    @pl.when(tid == 0)
    def _():
        pltpu.sync_copy(wk_v, sh_wk)
        pltpu.sync_copy(wi_v, sh_wi)
        pltpu.sync_copy(ws_v, sh_ws)

    plsc.subcore_barrier()

    pltpu.sync_copy(sh_cnt, acnt_v)
    gtv = zeros
    eqv = zeros
    for t in range(16):
        rowv = acnt_v[pl.ds(t * 16, 16)]
        gtv = gtv + jnp.where(iota == t, _extract(rowv, 0), 0)
        eqv = eqv + jnp.where(iota == t, _extract(rowv, 1), 0)
    total_gt = jnp.sum(gtv)
    ties = jnp.int32(K) - total_gt
    excl_eq = plsc.cumsum(eqv) - eqv
    quota_v = jnp.clip(ties - excl_eq, 0, eqv)
    taken_v = gtv + quota_v
    below = iota < tid
    base_eq_me = jnp.sum(jnp.where(below, eqv, 0))
    base_taken_me = jnp.sum(jnp.where(below, taken_v, 0))
    quota_me = jnp.clip(ties - base_eq_me, 0, neq)

    def _compact(j, c):
        eq_b, tk_b = c
        ku = key_v[pl.ds(j * 16, 16)]
        ks = ku ^ MININT
        m_gt = ks > thr_s
        m_eq = ku == thr
        eqr = eq_b + plsc.cumsum(jnp.where(m_eq, 1, 0)) - 1
        m_take = m_gt | (m_eq & (eqr < quota_me))
        mt = jnp.where(m_take, 1, 0)
        pos = tk_b + plsc.cumsum(mt) - 1
        gidx = base + j * 16 + iota
        plsc.store_scatter(wk_v, [pos], ku, mask=m_take)
        plsc.store_scatter(wi_v, [pos], gidx, mask=m_take)
        plsc.store_scatter(ws_v, [pos], sc_v[pl.ds(j * 16, 16)], mask=m_take)
        return eq_b + jnp.sum(jnp.where(m_eq, 1, 0)), tk_b + jnp.sum(mt)

    _, n_take = lax.fori_loop(0, VPT, _compact, (jnp.int32(0), jnp.int32(0)))

    def _pub(i, _):
        r = i * 16 + iota
        dest = jnp.where(r < n_take, base_taken_me + r,
                         (base_taken_me + r) & (K - 1))
        src = pl.ds(i * 16, 16)
        pltpu.sync_copy(wk_v.at[src], sh_wk.at[dest], add=True)
        pltpu.sync_copy(wi_v.at[src], sh_wi.at[dest], add=True)
        pltpu.sync_copy(ws_v.at[src], sh_ws.at[dest], add=True)
        return 0

    lax.fori_loop(0, (n_take + 15) // 16, _pub, 0)

    plsc.subcore_barrier()

    pltpu.sync_copy(sh_wk, awk_v)
    pltpu.sync_copy(sh_wi, awi_v)
    pltpu.sync_copy(sh_ws, aws_v)

    def _rank(jj, _):
        p = tid * WPT + jj
        krow = awk_v[pl.ds((p // 16) * 16, 16)]
        irow = awi_v[pl.ds((p // 16) * 16, 16)]
        lane = p % 16
        kj = _extract(krow, lane)
        ij = _extract(irow, lane)
        kjs = kj ^ MININT

        def _cmp(i, acc):
            av = awk_v[pl.ds(i * 16, 16)]
            asv = av ^ MININT
            beats = (asv > kjs) | ((av == kj) & (awi_v[pl.ds(i * 16, 16)] < ij))
            return acc + jnp.where(beats, 1, 0)

        accv = lax.fori_loop(0, K // 16, _cmp, zeros)
        rank = jnp.sum(accv)
        plsc.store_scatter(rank1_v, [jnp.full((16,), jj, jnp.int32)],
                           jnp.full((16,), rank, jnp.int32), mask=iota == 0)
        return 0

    lax.fori_loop(0, WPT, _rank, 0)

    for z in range(WPT // 16):
        myidx_v[pl.ds(z * 16, 16)] = awi_v[pl.ds(tid * WPT + z * 16, 16)]
        mysc_v[pl.ds(z * 16, 16)] = aws_v[pl.ds(tid * WPT + z * 16, 16)]

    pltpu.async_copy(node_hbm.at[myidx_v], rows_v, sem).wait()
    for z in range(WPT // 16):
        rk = rank1_v[pl.ds(z * 16, 16)]
        pltpu.async_copy(rows_v.at[pl.ds(z * 16, 16)], rows_hbm.at[rk],
                         sem).wait()
        pltpu.async_copy(mysc_v.at[pl.ds(z * 16, 16)], wscore_hbm.at[rk],
                         sem).wait()


_topk_call = functools.partial(
    pl.kernel,
    out_type=[
        jax.ShapeDtypeStruct((K, F), jnp.float32),
        jax.ShapeDtypeStruct((K,), jnp.float32),
    ],
    mesh=plsc.VectorSubcoreMesh(core_axis_name="c", subcore_axis_name="s",
                                num_cores=1),
    compiler_params=pltpu.CompilerParams(needs_layout_passes=False),
    scratch_types=[
        pltpu.VMEM((PER_TILE,), jnp.float32),
        pltpu.VMEM((PER_TILE,), jnp.int32),
        pltpu.VMEM((4096,), jnp.int32),
        pltpu.VMEM((256,), jnp.int32),
        pltpu.VMEM((4096,), jnp.int32),
        pltpu.VMEM((256,), jnp.int32),
        pltpu.VMEM((K,), jnp.int32),
        pltpu.VMEM((K,), jnp.int32),
        pltpu.VMEM((K,), jnp.float32),
        pltpu.VMEM((K,), jnp.int32),
        pltpu.VMEM((K,), jnp.int32),
        pltpu.VMEM((K,), jnp.float32),
        pltpu.VMEM((16,), jnp.int32),
        pltpu.VMEM((256,), jnp.int32),
        pltpu.VMEM((WPT,), jnp.int32),
        pltpu.VMEM((WPT,), jnp.int32),
        pltpu.VMEM((WPT,), jnp.float32),
        pltpu.VMEM((WPT, F), jnp.float32),
        pltpu.VMEM_SHARED((4, 4096), jnp.int32),
        pltpu.VMEM_SHARED((256,), jnp.int32),
        pltpu.VMEM_SHARED((K,), jnp.int32),
        pltpu.VMEM_SHARED((K,), jnp.int32),
        pltpu.VMEM_SHARED((K,), jnp.float32),
        pltpu.SemaphoreType.DMA,
    ],
)(_topk_body)


def _finish_body(rows_ref, ws_ref, o_ref):
    gated = rows_ref[...] * jnp.tanh(ws_ref[...])
    o_ref[...] = gated.T


_finish_call = pl.pallas_call(
    _finish_body,
    out_shape=jax.ShapeDtypeStruct((F, K), jnp.float32),
)


def kernel(node_embs, mask, W):
    scores = _scores_call(node_embs, mask, W).reshape(-1)
    rows, wscore = _topk_call(scores, node_embs)
    return _finish_call(rows, wscore.reshape(K, 1))


# speedup vs baseline: 1.1482x; 1.1482x over previous
import functools

import jax
import jax.numpy as jnp
from jax import lax
from jax.experimental import pallas as pl
from jax.experimental.pallas import tpu as pltpu
from jax.experimental.pallas import tpu_sc as plsc

N = 100000
F = 128
K = 1024

BLK = 2048
NBLK = 49
NPAD = BLK * NBLK
NT = 16
PER_TILE = NPAD // NT
VPT = PER_TILE // 16
WPT = K // NT
MININT = -2**31


def _scores_body(x_ref, m_ref, w_ref, o_ref):
    b = pl.program_id(0)
    w = w_ref[...]
    inv = 1.0 / jnp.sqrt(jnp.sum(w * w))
    s = jnp.dot(x_ref[...], w, preferred_element_type=jnp.float32) * inv
    s = s + m_ref[...]
    row = b * BLK + lax.broadcasted_iota(jnp.int32, (BLK, 1), 0)
    s = jnp.where(row < N, s, -jnp.inf)
    o_ref[...] = s.reshape(BLK // 128, 128)


_scores_call = pl.pallas_call(
    _scores_body,
    grid=(NBLK,),
    in_specs=[
        pl.BlockSpec((BLK, F), lambda i: (i, 0)),
        pl.BlockSpec((BLK, 1), lambda i: (i, 0)),
        pl.BlockSpec((F, 1), lambda i: (0, 0)),
    ],
    out_specs=pl.BlockSpec((BLK // 128, 128), lambda i: (i, 0)),
    out_shape=jax.ShapeDtypeStruct((NPAD // 128, 128), jnp.float32),
)


_IOTA = lambda: lax.iota(jnp.int32, 16)


def _extract(vec, i):
    return jnp.sum(jnp.where(_IOTA() == i, vec, 0))


def _sel16(vec, kk):
    rv = lax.rev(vec, (0,))
    cs = plsc.cumsum(rv)
    ffs = plsc.all_reduce_ffs(cs >= kk)
    if ffs.ndim:
        ffs = jnp.max(ffs)
    above = _extract(cs, ffs) - _extract(rv, ffs)
    return 15 - ffs, above


def _topk_body(scores_hbm, node_hbm, rows_hbm, wscore_hbm,
               sc_v, key_v, hist_v, ghl_v, ah_v, ghist_v,
               wk_v, wi_v, ws_v, awk_v, awi_v, aws_v,
               cnt_v, acnt_v, rank1_v, myidx_v, mysc_v, rows_v,
               sh_hist, sh_cnt, sh_wk, sh_wi, sh_ws, sem):
    tid = lax.axis_index("s")
    base = tid * PER_TILE
    iota = _IOTA()
    ones = jnp.ones((16,), jnp.int32)
    zeros = jnp.zeros((16,), jnp.int32)
    zerosf = jnp.zeros((16,), jnp.float32)

    pltpu.sync_copy(scores_hbm.at[pl.ds(base, PER_TILE)], sc_v)

    def _mkkeys(j, _):
        v = sc_v[pl.ds(j * 16, 16)]
        b = lax.bitcast_convert_type(v, jnp.int32)
        ku = jnp.where(b < 0, ~b, b ^ MININT)
        key_v[pl.ds(j * 16, 16)] = ku
        return 0

    lax.fori_loop(0, VPT, _mkkeys, 0)

    prefix = jnp.int32(0)
    kk = jnp.int32(K)
    for rnd in range(4):
        shift = 24 - 8 * rnd

        def _zero(g, _):
            hist_v[pl.ds(g * 16, 16)] = zeros
            return 0

        lax.fori_loop(0, 256, _zero, 0)

        pfx_hi = lax.shift_right_logical(prefix, shift + 8) if rnd else None

        def _hist(j, _):
            ku = key_v[pl.ds(j * 16, 16)]
            dig = lax.shift_right_logical(ku, shift) & 0xFF
            hidx = iota * 256 + dig
            if rnd == 0:
                plsc.addupdate_scatter(hist_v, [hidx], ones)
            else:
                m = lax.shift_right_logical(ku, shift + 8) == pfx_hi
                plsc.addupdate_scatter(hist_v, [hidx], ones, mask=m)
            return 0

        lax.fori_loop(0, VPT, _hist, 0)

        def _red(g, _):
            acc = zeros
            for l in range(16):
                acc = acc + hist_v[pl.ds(l * 256 + g * 16, 16)]
            ghl_v[pl.ds(g * 16, 16)] = acc
            return 0

        lax.fori_loop(0, 16, _red, 0)
        pltpu.sync_copy(ghl_v, sh_hist.at[rnd].at[pl.ds(tid * 256, 256)])
        plsc.subcore_barrier()
        pltpu.sync_copy(sh_hist.at[rnd], ah_v)

        def _gred(g, _):
            acc = zeros
            for t in range(16):
                acc = acc + ah_v[pl.ds(t * 256 + g * 16, 16)]
            ghist_v[pl.ds(g * 16, 16)] = acc
            return 0

        lax.fori_loop(0, 16, _gred, 0)

        sgv = zeros
        for g in range(16):
            sgv = sgv + jnp.where(iota == g,
                                  jnp.sum(ghist_v[pl.ds(g * 16, 16)]), 0)
        grp, above_g = _sel16(sgv, kk)
        gvec = ghist_v[pl.ds(grp * 16, 16)]
        dig, above_d = _sel16(gvec, kk - above_g)
        digit = grp * 16 + dig
        prefix = prefix | lax.shift_left(digit, shift)
        kk = kk - above_g - above_d

    thr = prefix
    thr_s = thr ^ MININT

    def _cnt(j, c):
        ku = key_v[pl.ds(j * 16, 16)]
        ks = ku ^ MININT
        cg, ce = c
        cg = cg + jnp.where(ks > thr_s, 1, 0)
        ce = ce + jnp.where(ku == thr, 1, 0)
        return cg, ce

    cgv, cev = lax.fori_loop(0, VPT, _cnt, (zeros, zeros))
    ngt = jnp.sum(cgv)
    neq = jnp.sum(cev)
    cnt_v[...] = jnp.where(iota == 0, ngt, 0) + jnp.where(iota == 1, neq, 0)

    def _zw(z, _):
        wk_v[pl.ds(z * 16, 16)] = zeros
        wi_v[pl.ds(z * 16, 16)] = zeros
        ws_v[pl.ds(z * 16, 16)] = zerosf
        return 0

    lax.fori_loop(0, K // 16, _zw, 0)

    pltpu.sync_copy(cnt_v, sh_cnt.at[pl.ds(tid * 16, 16)])

    @pl.when(tid == 0)
    def _():
        pltpu.sync_copy(wk_v, sh_wk)
        pltpu.sync_copy(wi_v, sh_wi)
        pltpu.sync_copy(ws_v, sh_ws)

    plsc.subcore_barrier()

    pltpu.sync_copy(sh_cnt, acnt_v)
    gtv = zeros
    eqv = zeros
    for t in range(16):
        rowv = acnt_v[pl.ds(t * 16, 16)]
        gtv = gtv + jnp.where(iota == t, _extract(rowv, 0), 0)
        eqv = eqv + jnp.where(iota == t, _extract(rowv, 1), 0)
    total_gt = jnp.sum(gtv)
    ties = jnp.int32(K) - total_gt
    excl_eq = plsc.cumsum(eqv) - eqv
    quota_v = jnp.clip(ties - excl_eq, 0, eqv)
    taken_v = gtv + quota_v
    below = iota < tid
    base_eq_me = jnp.sum(jnp.where(below, eqv, 0))
    base_taken_me = jnp.sum(jnp.where(below, taken_v, 0))
    quota_me = jnp.clip(ties - base_eq_me, 0, neq)

    def _compact(j, c):
        eq_b, tk_b = c
        ku = key_v[pl.ds(j * 16, 16)]
        ks = ku ^ MININT
        m_gt = ks > thr_s
        m_eq = ku == thr
        eqr = eq_b + plsc.cumsum(jnp.where(m_eq, 1, 0)) - 1
        m_take = m_gt | (m_eq & (eqr < quota_me))
        mt = jnp.where(m_take, 1, 0)
        pos = tk_b + plsc.cumsum(mt) - 1
        gidx = base + j * 16 + iota
        plsc.store_scatter(wk_v, [pos], ku, mask=m_take)
        plsc.store_scatter(wi_v, [pos], gidx, mask=m_take)
        plsc.store_scatter(ws_v, [pos], sc_v[pl.ds(j * 16, 16)], mask=m_take)
        return eq_b + jnp.sum(jnp.where(m_eq, 1, 0)), tk_b + jnp.sum(mt)

    _, n_take = lax.fori_loop(0, VPT, _compact, (jnp.int32(0), jnp.int32(0)))

    def _pub(i, _):
        r = i * 16 + iota
        dest = jnp.where(r < n_take, base_taken_me + r,
                         (base_taken_me + r) & (K - 1))
        src = pl.ds(i * 16, 16)
        pltpu.sync_copy(wk_v.at[src], sh_wk.at[dest], add=True)
        pltpu.sync_copy(wi_v.at[src], sh_wi.at[dest], add=True)
        pltpu.sync_copy(ws_v.at[src], sh_ws.at[dest], add=True)
        return 0

    lax.fori_loop(0, (n_take + 15) // 16, _pub, 0)

    plsc.subcore_barrier()

    pltpu.sync_copy(sh_wk, awk_v)
    pltpu.sync_copy(sh_wi, awi_v)
    pltpu.sync_copy(sh_ws, aws_v)

    def _rank(jj, _):
        p = tid * WPT + jj
        krow = awk_v[pl.ds((p // 16) * 16, 16)]
        irow = awi_v[pl.ds((p // 16) * 16, 16)]
        lane = p % 16
        kj = _extract(krow, lane)
        ij = _extract(irow, lane)
        kjs = kj ^ MININT

        def _cmp(i, acc):
            av = awk_v[pl.ds(i * 16, 16)]
            asv = av ^ MININT
            beats = (asv > kjs) | ((av == kj) & (awi_v[pl.ds(i * 16, 16)] < ij))
            return acc + jnp.where(beats, 1, 0)

        accv = lax.fori_loop(0, K // 16, _cmp, zeros)
        rank = jnp.sum(accv)
        plsc.store_scatter(rank1_v, [jnp.full((16,), jj, jnp.int32)],
                           jnp.full((16,), rank, jnp.int32), mask=iota == 0)
        return 0

    lax.fori_loop(0, WPT, _rank, 0)

    for z in range(WPT // 16):
        myidx_v[pl.ds(z * 16, 16)] = awi_v[pl.ds(tid * WPT + z * 16, 16)]
        mysc_v[pl.ds(z * 16, 16)] = aws_v[pl.ds(tid * WPT + z * 16, 16)]

    pltpu.async_copy(node_hbm.at[myidx_v], rows_v, sem).wait()
    for z in range(WPT // 16):
        rk = rank1_v[pl.ds(z * 16, 16)]
        pltpu.async_copy(rows_v.at[pl.ds(z * 16, 16)], rows_hbm.at[rk],
                         sem).wait()
        pltpu.async_copy(mysc_v.at[pl.ds(z * 16, 16)], wscore_hbm.at[rk],
                         sem).wait()


_topk_call = functools.partial(
    pl.kernel,
    out_type=[
        jax.ShapeDtypeStruct((K, F), jnp.float32),
        jax.ShapeDtypeStruct((K,), jnp.float32),
    ],
    mesh=plsc.VectorSubcoreMesh(core_axis_name="c", subcore_axis_name="s",
                                num_cores=1),
    compiler_params=pltpu.CompilerParams(needs_layout_passes=False),
    scratch_types=[
        pltpu.VMEM((PER_TILE,), jnp.float32),
        pltpu.VMEM((PER_TILE,), jnp.int32),
        pltpu.VMEM((4096,), jnp.int32),
        pltpu.VMEM((256,), jnp.int32),
        pltpu.VMEM((4096,), jnp.int32),
        pltpu.VMEM((256,), jnp.int32),
        pltpu.VMEM((K,), jnp.int32),
        pltpu.VMEM((K,), jnp.int32),
        pltpu.VMEM((K,), jnp.float32),
        pltpu.VMEM((K,), jnp.int32),
        pltpu.VMEM((K,), jnp.int32),
        pltpu.VMEM((K,), jnp.float32),
        pltpu.VMEM((16,), jnp.int32),
        pltpu.VMEM((256,), jnp.int32),
        pltpu.VMEM((WPT,), jnp.int32),
        pltpu.VMEM((WPT,), jnp.int32),
        pltpu.VMEM((WPT,), jnp.float32),
        pltpu.VMEM((WPT, F), jnp.float32),
        pltpu.VMEM_SHARED((4, 4096), jnp.int32),
        pltpu.VMEM_SHARED((256,), jnp.int32),
        pltpu.VMEM_SHARED((K,), jnp.int32),
        pltpu.VMEM_SHARED((K,), jnp.int32),
        pltpu.VMEM_SHARED((K,), jnp.float32),
        pltpu.SemaphoreType.DMA,
    ],
)(_topk_body)


def _finish_body(rows_ref, ws_ref, o_ref):
    gated = rows_ref[...] * jnp.tanh(ws_ref[...])
    o_ref[...] = gated.T


_finish_call = pl.pallas_call(
    _finish_body,
    out_shape=jax.ShapeDtypeStruct((F, K), jnp.float32),
)


def kernel(node_embs, mask, W):
    scores = _scores_call(node_embs, mask, W).reshape(-1)
    rows, wscore = _topk_call(scores, node_embs)
    return _finish_call(rows, wscore.reshape(K, 1))
